# Initial kernel scaffold; baseline (speedup 1.0000x reference)
import jax
import jax.numpy as jnp
from jax.experimental import pallas as pl


def kernel(source, target, table):
    raise NotImplementedError("write your pallas kernel here")


# speedup vs baseline: 1.0733x; 1.0733x over previous
import functools

import jax
import jax.numpy as jnp
from jax import lax
from jax.experimental import pallas as pl
from jax.experimental.pallas import tpu as pltpu
from jax.experimental.pallas import tpu_sc as plsc

IN_DIM = 1000000
HID_C = 32

NC = 2
NS = 16
NL = 16
NW = NC * NS

CHUNK = 128


def _make_gather(n_total: int):
    assert n_total % (NW * CHUNK) == 0
    n_per_w = n_total // NW
    chunks_per_w = n_per_w // CHUNK
    mesh = plsc.VectorSubcoreMesh(core_axis_name="c", subcore_axis_name="s")

    @functools.partial(
        pl.kernel,
        mesh=mesh,
        out_type=jax.ShapeDtypeStruct((n_total, HID_C), jnp.float32),
        compiler_params=pltpu.CompilerParams(
            needs_layout_passes=False, use_tc_tiling_on_sc=False
        ),
        scratch_types=[
            pltpu.VMEM((CHUNK,), jnp.int32),
            pltpu.VMEM((CHUNK, HID_C), jnp.float32),
            pltpu.SemaphoreType.DMA,
        ],
    )
    def gather_kernel(src_hbm, table_hbm, out_hbm, idx_v, rows_v, sem):
        wid = lax.axis_index("s") * NC + lax.axis_index("c")
        base = wid * n_per_w

        @pl.loop(0, chunks_per_w)
        def _(ci):
            off = base + ci * CHUNK
            pltpu.sync_copy(src_hbm.at[pl.ds(off, CHUNK)], idx_v)

            @pl.loop(0, CHUNK, step=NL)
            def _(j):
                v = idx_v[pl.ds(j, NL)]
                s = v // 20 - 1
                s = jnp.where(s < 0, s + IN_DIM, s)
                idx_v[pl.ds(j, NL)] = s

            pltpu.async_copy(table_hbm.at[idx_v], rows_v, sem).wait()
            pltpu.sync_copy(rows_v, out_hbm.at[pl.ds(off, CHUNK)])

    return gather_kernel


def _t_body(tgt_ref, out_ref):
    out_ref[...] = tgt_ref[...] // 20 - 1


@jax.jit
def kernel(source, target, table):
    b, seq = source.shape
    n_total = b * seq
    emb_flat = _make_gather(n_total)(source.reshape(n_total), table)
    t = pl.pallas_call(
        _t_body,
        out_shape=jax.ShapeDtypeStruct(target.shape, target.dtype),
    )(target)
    return (emb_flat.reshape(b, seq, HID_C), t)


# speedup vs baseline: 1.1787x; 1.0982x over previous
import functools

import jax
import jax.numpy as jnp
from jax.experimental import pallas as pl
from jax.experimental.pallas import tpu as pltpu
from jax.experimental.pallas import tpu_sc as plsc

IN_DIM = 1000000
HID_C = 32

NL = 16
GW = 128
G = 10
W = G * GW


def _make_gather(n_total: int):
    assert n_total % W == 0
    mesh = plsc.VectorSubcoreMesh(core_axis_name="c", subcore_axis_name="s")

    @functools.partial(
        pl.kernel,
        mesh=mesh,
        out_type=jax.ShapeDtypeStruct((n_total, HID_C), jnp.float32),
        compiler_params=pltpu.CompilerParams(
            needs_layout_passes=False, use_tc_tiling_on_sc=False
        ),
        scratch_types=[
            pltpu.VMEM((G, GW), jnp.int32),
            pltpu.SemaphoreType.DMA,
        ],
    )
    def gather_kernel(src_hbm, table_hbm, out_hbm, idx_s, sem):
        def body(src_vmem, out_vmem):
            for g in range(G):
                for j in range(GW // NL):
                    v = src_vmem[0, pl.ds(g * GW + j * NL, NL)]
                    s = v // 20 - 1
                    s = jnp.where(s < 0, s + IN_DIM, s)
                    idx_s[g, pl.ds(j * NL, NL)] = s
            copies = [
                pltpu.async_copy(
                    table_hbm.at[idx_s.at[g]],
                    out_vmem.at[pl.ds(g * GW, GW)],
                    sem,
                )
                for g in range(G)
            ]
            for c in copies:
                c.wait()

        pltpu.emit_pipeline(
            body,
            grid=(n_total // W,),
            in_specs=[pl.BlockSpec((1, W), index_map=lambda i: (0, i))],
            out_specs=[pl.BlockSpec((W, HID_C), index_map=lambda i: (i, 0))],
            core_axis_name=("c", "s"),
            dimension_semantics=(pltpu.PARALLEL,),
        )(src_hbm, out_hbm)

    return gather_kernel


def _t_body(tgt_ref, out_ref):
    out_ref[...] = tgt_ref[...] // 20 - 1


@jax.jit
def kernel(source, target, table):
    b, seq = source.shape
    n_total = b * seq
    emb_flat = _make_gather(n_total)(source.reshape(1, n_total), table)
    t = pl.pallas_call(
        _t_body,
        out_shape=jax.ShapeDtypeStruct(target.shape, target.dtype),
    )(target)
    return (emb_flat.reshape(b, seq, HID_C), t)


# speedup vs baseline: 1.3266x; 1.1255x over previous
import functools

import jax
import jax.numpy as jnp
from jax import lax
from jax.experimental import pallas as pl
from jax.experimental.pallas import tpu as pltpu
from jax.experimental.pallas import tpu_sc as plsc

IN_DIM = 1000000
HID_C = 32

GW = 128
G = 8
W = G * GW

LB = 8


def _prep_body(src_ref, tgt_ref, idx_ref, t_ref):
    v = src_ref[...]
    s = v // 20 - 1
    s = jnp.where(s < 0, s + IN_DIM, s)
    for l in range(LB):
        for q in range(32):
            idx_ref[32 * l + q : 32 * l + q + 1, :] = (
                s[l : l + 1, 128 * q : 128 * (q + 1)]
            )
    t_ref[...] = tgt_ref[...] // 20 - 1


def _out_body(rows_ref, emb_ref):
    x = rows_ref[...]
    xt = jnp.transpose(x)
    for u in range(4):
        emb_ref[0, :, 256 * u : 256 * (u + 1)] = xt[32 * u : 32 * (u + 1), :]


def _make_gather(n_total: int):
    assert n_total % W == 0
    mesh = plsc.VectorSubcoreMesh(core_axis_name="c", subcore_axis_name="s")

    @functools.partial(
        pl.kernel,
        mesh=mesh,
        out_type=jax.ShapeDtypeStruct((n_total, HID_C), jnp.float32),
        compiler_params=pltpu.CompilerParams(
            needs_layout_passes=False, use_tc_tiling_on_sc=False
        ),
        scratch_types=[
            pltpu.VMEM((G, GW), jnp.int32),
            pltpu.SemaphoreType.DMA,
        ],
    )
    def gather_kernel(idx_hbm, table_hbm, out_hbm, idx2, sem):
        def body(idx_vmem, out_vmem):
            for w in range(G):
                for k in range(GW // 16):
                    c = lax.iota(jnp.int32, 16) + 16 * k
                    rowv = 2 * (c % 4) + (w // 4)
                    colv = 32 * (w % 4) + c // 4
                    idx2[w, pl.ds(16 * k, 16)] = plsc.load_gather(
                        idx_vmem, [rowv, colv]
                    )
            copies = [
                pltpu.async_copy(
                    table_hbm.at[idx2.at[g]],
                    out_vmem.at[pl.ds(g * GW, GW)],
                    sem,
                )
                for g in range(G)
            ]
            for cp in copies:
                cp.wait()

        pltpu.emit_pipeline(
            body,
            grid=(n_total // W,),
            in_specs=[pl.BlockSpec((G, GW), index_map=lambda i: (i, 0))],
            out_specs=[pl.BlockSpec((W, HID_C), index_map=lambda i: (i, 0))],
            core_axis_name=("c", "s"),
            dimension_semantics=(pltpu.PARALLEL,),
        )(idx_hbm, out_hbm)

    return gather_kernel


@jax.jit
def kernel(source, target, table):
    b, seq = source.shape
    n_total = b * seq

    src_t = source.T
    tgt_t = target.T

    idx_flat, t_t = pl.pallas_call(
        _prep_body,
        grid=(seq // LB,),
        in_specs=[
            pl.BlockSpec((LB, b), lambda i: (i, 0)),
            pl.BlockSpec((LB, b), lambda i: (i, 0)),
        ],
        out_specs=[
            pl.BlockSpec((LB * b // 128, 128), lambda i: (i, 0)),
            pl.BlockSpec((LB, b), lambda i: (i, 0)),
        ],
        out_shape=[
            jax.ShapeDtypeStruct((n_total // 128, 128), jnp.int32),
            jax.ShapeDtypeStruct((seq, b), target.dtype),
        ],
    )(src_t, tgt_t)

    rows = _make_gather(n_total)(idx_flat, table)

    emb_t = pl.pallas_call(
        _out_body,
        grid=(seq, 4),
        in_specs=[pl.BlockSpec((W * HID_C // 128, 128), lambda i, j: (4 * i + j, 0))],
        out_specs=pl.BlockSpec((1, HID_C, b // 4), lambda i, j: (i, 0, j)),
        out_shape=jax.ShapeDtypeStruct((seq, HID_C, b), jnp.float32),
    )(rows.reshape(n_total * HID_C // 128, 128))

    return (jnp.transpose(emb_t, (2, 0, 1)), t_t.T)


# speedup vs baseline: 2.4615x; 1.8555x over previous
import functools

import jax
import jax.numpy as jnp
from jax import lax
from jax.experimental import pallas as pl
from jax.experimental.pallas import tpu as pltpu
from jax.experimental.pallas import tpu_sc as plsc

IN_DIM = 1000000
HID_C = 32

GW = 128
G = 8
W = G * GW

LB = 8
VC = 4096


def _prep_body(src_ref, tgt_ref, idx_ref, t_ref):
    v = src_ref[...]
    s = v // 20 - 1
    s = jnp.where(s < 0, s + IN_DIM, s)
    rem = s % VC
    s = (s - rem) + 4 * (s % 1024) + rem // 1024
    for l in range(LB):
        for q in range(32):
            idx_ref[32 * l + q : 32 * l + q + 1, :] = (
                s[l : l + 1, 128 * q : 128 * (q + 1)]
            )
    t_ref[...] = tgt_ref[...] // 20 - 1


def _tab_body(tab_ref, w_ref):
    x = tab_ref[...]
    xt = jnp.transpose(x)
    for u in range(4):
        w_ref[:, HID_C * u : HID_C * (u + 1)] = xt[
            (VC // 4) * u : (VC // 4) * (u + 1), :
        ]


def _out_body(rows_ref, emb_ref):
    x = rows_ref[...]
    for q in range(4):
        xt = jnp.transpose(x[256 * q : 256 * (q + 1), :])
        for u in range(4):
            emb_ref[0, :, 1024 * q + 256 * u : 1024 * q + 256 * (u + 1)] = xt[
                32 * u : 32 * (u + 1), :
            ]


def _make_gather(n_total: int, v_rows: int):
    assert n_total % W == 0
    mesh = plsc.VectorSubcoreMesh(core_axis_name="c", subcore_axis_name="s")

    @functools.partial(
        pl.kernel,
        mesh=mesh,
        out_type=jax.ShapeDtypeStruct((n_total, HID_C), jnp.float32),
        compiler_params=pltpu.CompilerParams(
            needs_layout_passes=False, use_tc_tiling_on_sc=False
        ),
        scratch_types=[
            pltpu.VMEM((G, GW), jnp.int32),
            pltpu.SemaphoreType.DMA,
        ],
    )
    def gather_kernel(idx_hbm, table_hbm, out_hbm, idx2, sem):
        def body(idx_vmem, out_vmem):
            for w in range(G):
                for k in range(GW // 16):
                    c = lax.iota(jnp.int32, 16) + 16 * k
                    rowv = 2 * (c % 4) + (w // 4)
                    colv = 32 * (w % 4) + c // 4
                    idx2[w, pl.ds(16 * k, 16)] = plsc.load_gather(
                        idx_vmem, [rowv, colv]
                    )
            copies = [
                pltpu.async_copy(
                    table_hbm.at[idx2.at[g]],
                    out_vmem.at[pl.ds(g * GW, GW)],
                    sem,
                )
                for g in range(G)
            ]
            for cp in copies:
                cp.wait()

        pltpu.emit_pipeline(
            body,
            grid=(n_total // W,),
            in_specs=[pl.BlockSpec((G, GW), index_map=lambda i: (i, 0))],
            out_specs=[pl.BlockSpec((W, HID_C), index_map=lambda i: (i, 0))],
            core_axis_name=("c", "s"),
            dimension_semantics=(pltpu.PARALLEL,),
        )(idx_hbm, out_hbm)

    return gather_kernel


@jax.jit
def kernel(source, target, table):
    b, seq = source.shape
    n_total = b * seq
    v_dim = table.shape[0]
    nvb = (v_dim + VC - 1) // VC

    src_t = source.T
    tgt_t = target.T

    idx_flat, t_t = pl.pallas_call(
        _prep_body,
        grid=(seq // LB,),
        in_specs=[
            pl.BlockSpec((LB, b), lambda i: (i, 0)),
            pl.BlockSpec((LB, b), lambda i: (i, 0)),
        ],
        out_specs=[
            pl.BlockSpec((LB * b // 128, 128), lambda i: (i, 0)),
            pl.BlockSpec((LB, b), lambda i: (i, 0)),
        ],
        out_shape=[
            jax.ShapeDtypeStruct((n_total // 128, 128), jnp.int32),
            jax.ShapeDtypeStruct((seq, b), target.dtype),
        ],
    )(src_t, tgt_t)

    w_tab = pl.pallas_call(
        _tab_body,
        grid=(nvb,),
        in_specs=[pl.BlockSpec((HID_C, VC), lambda i: (0, i))],
        out_specs=pl.BlockSpec((VC // 4, 128), lambda i: (i, 0)),
        out_shape=jax.ShapeDtypeStruct((nvb * VC // 4, 128), jnp.float32),
    )(table.T)

    rows = _make_gather(n_total, nvb * VC)(
        idx_flat, w_tab.reshape(nvb * VC, HID_C)
    )

    emb_t = pl.pallas_call(
        _out_body,
        grid=(seq,),
        in_specs=[pl.BlockSpec((b * HID_C // 128, 128), lambda i: (i, 0))],
        out_specs=pl.BlockSpec((1, HID_C, b), lambda i: (i, 0, 0)),
        out_shape=jax.ShapeDtypeStruct((seq, HID_C, b), jnp.float32),
    )(rows.reshape(n_total * HID_C // 128, 128))

    return (jnp.transpose(emb_t, (2, 0, 1)), t_t.T)


# speedup vs baseline: 2.4649x; 1.0014x over previous
import functools

import jax
import jax.numpy as jnp
from jax import lax
from jax.experimental import pallas as pl
from jax.experimental.pallas import tpu as pltpu
from jax.experimental.pallas import tpu_sc as plsc

IN_DIM = 1000000
HID_C = 32

GW = 128
G = 8
W = G * GW

LB = 8
VC = 4096


def _prep_body(src_ref, tgt_ref, idx_ref, t_ref):
    v = src_ref[...]
    s = v // 20 - 1
    s = jnp.where(s < 0, s + IN_DIM, s)
    rem = s % VC
    s = (s - rem) + 4 * (s % 1024) + rem // 1024
    for l in range(LB):
        for q in range(32):
            idx_ref[32 * l + q : 32 * l + q + 1, :] = (
                s[l : l + 1, 128 * q : 128 * (q + 1)]
            )
    t_ref[...] = tgt_ref[...] // 20 - 1


def _tab_body(tab_ref, w_ref):
    x = tab_ref[...]
    xt = jnp.transpose(x)
    for u in range(4):
        w_ref[:, HID_C * u : HID_C * (u + 1)] = xt[
            (VC // 4) * u : (VC // 4) * (u + 1), :
        ]


def _out_body(rows_ref, emb_ref):
    x = rows_ref[...]
    for q in range(4):
        xt = jnp.transpose(x[256 * q : 256 * (q + 1), :])
        for u in range(4):
            emb_ref[0, :, 1024 * q + 256 * u : 1024 * q + 256 * (u + 1)] = xt[
                32 * u : 32 * (u + 1), :
            ]


def _make_gather(n_total: int, v_rows: int):
    assert n_total % W == 0
    mesh = plsc.VectorSubcoreMesh(core_axis_name="c", subcore_axis_name="s")

    @functools.partial(
        pl.kernel,
        mesh=mesh,
        out_type=jax.ShapeDtypeStruct((n_total, HID_C), jnp.float32),
        compiler_params=pltpu.CompilerParams(
            needs_layout_passes=False, use_tc_tiling_on_sc=False
        ),
        scratch_types=[
            pltpu.VMEM((G, GW), jnp.int32),
            pltpu.SemaphoreType.DMA,
        ],
    )
    def gather_kernel(idx_hbm, table_hbm, out_hbm, idx2, sem):
        def body(idx_vmem, out_vmem):
            for w in range(G):
                for k in range(GW // 16):
                    c = lax.iota(jnp.int32, 16) + 16 * k
                    rowv = 2 * (c % 4) + (w // 4)
                    colv = 32 * (w % 4) + c // 4
                    idx2[w, pl.ds(16 * k, 16)] = plsc.load_gather(
                        idx_vmem, [rowv, colv]
                    )
            copies = [
                pltpu.async_copy(
                    table_hbm.at[idx2.at[g]],
                    out_vmem.at[pl.ds(g * GW, GW)],
                    sem,
                )
                for g in range(G)
            ]
            for cp in copies:
                cp.wait()

        pltpu.emit_pipeline(
            body,
            grid=(n_total // W,),
            in_specs=[pl.BlockSpec((G, GW), index_map=lambda i: (i, 0))],
            out_specs=[pl.BlockSpec((W, HID_C), index_map=lambda i: (i, 0))],
            core_axis_name=("c", "s"),
            dimension_semantics=(pltpu.PARALLEL,),
        )(idx_hbm, out_hbm)

    return gather_kernel


@jax.jit
def kernel(source, target, table):
    b, seq = source.shape
    n_total = b * seq
    v_dim = table.shape[0]
    nvb = (v_dim + VC - 1) // VC

    src_t = source.T
    tgt_t = target.T

    idx_flat, t_t = pl.pallas_call(
        _prep_body,
        grid=(seq // LB,),
        in_specs=[
            pl.BlockSpec((LB, b), lambda i: (i, 0)),
            pl.BlockSpec((LB, b), lambda i: (i, 0)),
        ],
        out_specs=[
            pl.BlockSpec((LB * b // 128, 128), lambda i: (i, 0)),
            pl.BlockSpec((LB, b), lambda i: (i, 0)),
        ],
        out_shape=[
            jax.ShapeDtypeStruct((n_total // 128, 128), jnp.int32),
            jax.ShapeDtypeStruct((seq, b), target.dtype),
        ],
        compiler_params=pltpu.CompilerParams(
            dimension_semantics=("parallel",)
        ),
    )(src_t, tgt_t)

    w_tab = pl.pallas_call(
        _tab_body,
        grid=(nvb,),
        in_specs=[pl.BlockSpec((HID_C, VC), lambda i: (0, i))],
        out_specs=pl.BlockSpec((VC // 4, 128), lambda i: (i, 0)),
        out_shape=jax.ShapeDtypeStruct((nvb * VC // 4, 128), jnp.float32),
        compiler_params=pltpu.CompilerParams(
            dimension_semantics=("parallel",)
        ),
    )(table.T)

    rows = _make_gather(n_total, nvb * VC)(
        idx_flat, w_tab.reshape(nvb * VC, HID_C)
    )

    emb_t = pl.pallas_call(
        _out_body,
        grid=(seq,),
        in_specs=[pl.BlockSpec((b * HID_C // 128, 128), lambda i: (i, 0))],
        out_specs=pl.BlockSpec((1, HID_C, b), lambda i: (i, 0, 0)),
        out_shape=jax.ShapeDtypeStruct((seq, HID_C, b), jnp.float32),
        compiler_params=pltpu.CompilerParams(
            dimension_semantics=("parallel",)
        ),
    )(rows.reshape(n_total * HID_C // 128, 128))

    return (jnp.transpose(emb_t, (2, 0, 1)), t_t.T)


# speedup vs baseline: 2.6768x; 1.0860x over previous
import functools

import jax
import jax.numpy as jnp
from jax import lax
from jax.experimental import pallas as pl
from jax.experimental.pallas import tpu as pltpu
from jax.experimental.pallas import tpu_sc as plsc

IN_DIM = 1000000
HID_C = 32

GW = 128
G = 8
W = G * GW

LB = 8
VC = 4096


def _prep_body(src_ref, tgt_ref, idx_ref, t_ref):
    v = src_ref[...]
    s = v // 20 - 1
    s = jnp.where(s < 0, s + IN_DIM, s)
    rem = s % VC
    s = (s - rem) + 4 * (s % 1024) + rem // 1024
    for l in range(LB):
        for q in range(32):
            idx_ref[32 * l + q : 32 * l + q + 1, :] = (
                s[l : l + 1, 128 * q : 128 * (q + 1)]
            )
    t_ref[...] = tgt_ref[...] // 20 - 1


def _tab_body(tab_ref, w_ref):
    x = tab_ref[...]
    w_ref[...] = jnp.concatenate(
        [
            jnp.transpose(x[:, (VC // 4) * u : (VC // 4) * (u + 1)])
            for u in range(4)
        ],
        axis=1,
    )


OLB = 2


def _out_body(rows_ref, emb_ref):
    for p in range(OLB):
        x = rows_ref[1024 * p : 1024 * (p + 1), :]
        for q in range(4):
            xt = jnp.transpose(x[256 * q : 256 * (q + 1), :])
            for u in range(4):
                emb_ref[
                    p, :, 1024 * q + 256 * u : 1024 * q + 256 * (u + 1)
                ] = xt[32 * u : 32 * (u + 1), :]


def _make_gather(n_total: int, v_rows: int):
    assert n_total % W == 0
    mesh = plsc.VectorSubcoreMesh(core_axis_name="c", subcore_axis_name="s")

    @functools.partial(
        pl.kernel,
        mesh=mesh,
        out_type=jax.ShapeDtypeStruct((n_total, HID_C), jnp.float32),
        compiler_params=pltpu.CompilerParams(
            needs_layout_passes=False, use_tc_tiling_on_sc=False
        ),
        scratch_types=[
            pltpu.VMEM((G, GW), jnp.int32),
            pltpu.SemaphoreType.DMA,
        ],
    )
    def gather_kernel(idx_hbm, table_hbm, out_hbm, idx2, sem):
        def body(idx_vmem, out_vmem):
            for w in range(G):
                for k in range(GW // 16):
                    c = lax.iota(jnp.int32, 16) + 16 * k
                    rowv = 2 * (c % 4) + (w // 4)
                    colv = 32 * (w % 4) + c // 4
                    idx2[w, pl.ds(16 * k, 16)] = plsc.load_gather(
                        idx_vmem, [rowv, colv]
                    )
            copies = [
                pltpu.async_copy(
                    table_hbm.at[idx2.at[g]],
                    out_vmem.at[pl.ds(g * GW, GW)],
                    sem,
                )
                for g in range(G)
            ]
            for cp in copies:
                cp.wait()

        pltpu.emit_pipeline(
            body,
            grid=(n_total // W,),
            in_specs=[pl.BlockSpec((G, GW), index_map=lambda i: (i, 0))],
            out_specs=[pl.BlockSpec((W, HID_C), index_map=lambda i: (i, 0))],
            core_axis_name=("c", "s"),
            dimension_semantics=(pltpu.PARALLEL,),
        )(idx_hbm, out_hbm)

    return gather_kernel


@jax.jit
def kernel(source, target, table):
    b, seq = source.shape
    n_total = b * seq
    v_dim = table.shape[0]
    nvb = (v_dim + VC - 1) // VC

    src_t = source.T
    tgt_t = target.T

    idx_flat, t_t = pl.pallas_call(
        _prep_body,
        grid=(seq // LB,),
        in_specs=[
            pl.BlockSpec((LB, b), lambda i: (i, 0)),
            pl.BlockSpec((LB, b), lambda i: (i, 0)),
        ],
        out_specs=[
            pl.BlockSpec((LB * b // 128, 128), lambda i: (i, 0)),
            pl.BlockSpec((LB, b), lambda i: (i, 0)),
        ],
        out_shape=[
            jax.ShapeDtypeStruct((n_total // 128, 128), jnp.int32),
            jax.ShapeDtypeStruct((seq, b), target.dtype),
        ],
        compiler_params=pltpu.CompilerParams(
            dimension_semantics=("parallel",)
        ),
    )(src_t, tgt_t)

    w_tab = pl.pallas_call(
        _tab_body,
        grid=(nvb,),
        in_specs=[pl.BlockSpec((HID_C, VC), lambda i: (0, i))],
        out_specs=pl.BlockSpec((VC // 4, 128), lambda i: (i, 0)),
        out_shape=jax.ShapeDtypeStruct((nvb * VC // 4, 128), jnp.float32),
        compiler_params=pltpu.CompilerParams(
            dimension_semantics=("parallel",)
        ),
    )(table.T)

    rows = _make_gather(n_total, nvb * VC)(
        idx_flat, w_tab.reshape(nvb * VC, HID_C)
    )

    emb_t = pl.pallas_call(
        _out_body,
        grid=(seq // OLB,),
        in_specs=[pl.BlockSpec((OLB * b * HID_C // 128, 128), lambda i: (i, 0))],
        out_specs=pl.BlockSpec((OLB, HID_C, b), lambda i: (i, 0, 0)),
        out_shape=jax.ShapeDtypeStruct((seq, HID_C, b), jnp.float32),
        compiler_params=pltpu.CompilerParams(
            dimension_semantics=("parallel",)
        ),
    )(rows.reshape(n_total * HID_C // 128, 128))

    return (jnp.transpose(emb_t, (2, 0, 1)), t_t.T)


# speedup vs baseline: 3.2884x; 1.2285x over previous
import functools

import jax
import jax.numpy as jnp
from jax import lax
from jax.experimental import pallas as pl
from jax.experimental.pallas import tpu as pltpu
from jax.experimental.pallas import tpu_sc as plsc

IN_DIM = 1000000
HID_C = 32

GW = 128
G = 8
W = G * GW

LB = 8
VC = 4096


def _prep_body(src_ref, tgt_ref, idx_ref, t_ref):
    v = src_ref[...]
    s = v // 20 - 1
    s = jnp.where(s < 0, s + IN_DIM, s)
    rem = s % VC
    s = (s - rem) + 4 * (s % 1024) + rem // 1024
    for l in range(LB):
        for q in range(32):
            idx_ref[32 * l + q : 32 * l + q + 1, :] = (
                s[l : l + 1, 128 * q : 128 * (q + 1)]
            )
    t_ref[...] = tgt_ref[...] // 20 - 1


def _tab_body(tab_ref, w_ref):
    x = tab_ref[...]
    z = jnp.concatenate(
        [x[:, (VC // 4) * u : (VC // 4) * (u + 1)] for u in range(4)], axis=0
    )
    w_ref[...] = jnp.transpose(z)


OLB = 2


def _out_body(rows_ref, emb_ref):
    for p in range(OLB):
        x = rows_ref[1024 * p : 1024 * (p + 1), :]
        for q in range(4):
            xt = jnp.transpose(x[256 * q : 256 * (q + 1), :])
            for u in range(4):
                emb_ref[
                    p, :, 1024 * q + 256 * u : 1024 * q + 256 * (u + 1)
                ] = xt[32 * u : 32 * (u + 1), :]


def _make_gather(n_total: int, v_rows: int):
    assert n_total % W == 0
    mesh = plsc.VectorSubcoreMesh(core_axis_name="c", subcore_axis_name="s")

    @functools.partial(
        pl.kernel,
        mesh=mesh,
        out_type=jax.ShapeDtypeStruct((n_total, HID_C), jnp.float32),
        compiler_params=pltpu.CompilerParams(
            needs_layout_passes=False, use_tc_tiling_on_sc=False
        ),
        scratch_types=[
            pltpu.VMEM((G, GW), jnp.int32),
            pltpu.SemaphoreType.DMA,
        ],
    )
    def gather_kernel(idx_hbm, table_hbm, out_hbm, idx2, sem):
        def body(idx_vmem, out_vmem):
            for w in range(G):
                for k in range(GW // 16):
                    c = lax.iota(jnp.int32, 16) + 16 * k
                    rowv = 2 * (c % 4) + (w // 4)
                    colv = 32 * (w % 4) + c // 4
                    idx2[w, pl.ds(16 * k, 16)] = plsc.load_gather(
                        idx_vmem, [rowv, colv]
                    )
            copies = [
                pltpu.async_copy(
                    table_hbm.at[idx2.at[g]],
                    out_vmem.at[pl.ds(g * GW, GW)],
                    sem,
                )
                for g in range(G)
            ]
            for cp in copies:
                cp.wait()

        pltpu.emit_pipeline(
            body,
            grid=(n_total // W,),
            in_specs=[pl.BlockSpec((G, GW), index_map=lambda i: (i, 0))],
            out_specs=[pl.BlockSpec((W, HID_C), index_map=lambda i: (i, 0))],
            core_axis_name=("c", "s"),
            dimension_semantics=(pltpu.PARALLEL,),
        )(idx_hbm, out_hbm)

    return gather_kernel


@jax.jit
def kernel(source, target, table):
    b, seq = source.shape
    n_total = b * seq
    v_dim = table.shape[0]
    nvb = (v_dim + VC - 1) // VC

    src_t = source.T
    tgt_t = target.T

    idx_flat, t_t = pl.pallas_call(
        _prep_body,
        grid=(seq // LB,),
        in_specs=[
            pl.BlockSpec((LB, b), lambda i: (i, 0)),
            pl.BlockSpec((LB, b), lambda i: (i, 0)),
        ],
        out_specs=[
            pl.BlockSpec((LB * b // 128, 128), lambda i: (i, 0)),
            pl.BlockSpec((LB, b), lambda i: (i, 0)),
        ],
        out_shape=[
            jax.ShapeDtypeStruct((n_total // 128, 128), jnp.int32),
            jax.ShapeDtypeStruct((seq, b), target.dtype),
        ],
        compiler_params=pltpu.CompilerParams(
            dimension_semantics=("parallel",)
        ),
    )(src_t, tgt_t)

    w_tab = pl.pallas_call(
        _tab_body,
        grid=(nvb,),
        in_specs=[pl.BlockSpec((HID_C, VC), lambda i: (0, i))],
        out_specs=pl.BlockSpec((VC // 4, 128), lambda i: (i, 0)),
        out_shape=jax.ShapeDtypeStruct((nvb * VC // 4, 128), jnp.float32),
        compiler_params=pltpu.CompilerParams(
            dimension_semantics=("parallel",)
        ),
    )(table.T)

    rows = _make_gather(n_total, nvb * VC)(
        idx_flat, w_tab.reshape(nvb * VC, HID_C)
    )

    emb_t = pl.pallas_call(
        _out_body,
        grid=(seq // OLB,),
        in_specs=[pl.BlockSpec((OLB * b * HID_C // 128, 128), lambda i: (i, 0))],
        out_specs=pl.BlockSpec((OLB, HID_C, b), lambda i: (i, 0, 0)),
        out_shape=jax.ShapeDtypeStruct((seq, HID_C, b), jnp.float32),
        compiler_params=pltpu.CompilerParams(
            dimension_semantics=("parallel",)
        ),
    )(rows.reshape(n_total * HID_C // 128, 128))

    return (jnp.transpose(emb_t, (2, 0, 1)), t_t.T)


# speedup vs baseline: 4.1264x; 1.2548x over previous
import functools

import jax
import jax.numpy as jnp
from jax import lax
from jax.experimental import pallas as pl
from jax.experimental.pallas import tpu as pltpu
from jax.experimental.pallas import tpu_sc as plsc

IN_DIM = 1000000
HID_C = 32

GW = 128
G = 8
W = G * GW

LB = 8
VC = 8192


def _prep_body(src_ref, tgt_ref, idx_ref, t_ref):
    v = src_ref[...]
    s = v // 20 - 1
    s = jnp.where(s < 0, s + IN_DIM, s)
    rem = s % VC
    s = (s - rem) + 4 * (rem % (VC // 4)) + rem // (VC // 4)
    for l in range(LB):
        for q in range(32):
            idx_ref[32 * l + q : 32 * l + q + 1, :] = (
                s[l : l + 1, 128 * q : 128 * (q + 1)]
            )
    t_ref[...] = tgt_ref[...] // 20 - 1


def _tab_body(tab_ref, w_ref):
    x = tab_ref[...]
    z = jnp.concatenate(
        [x[:, (VC // 4) * u : (VC // 4) * (u + 1)] for u in range(4)], axis=0
    )
    w_ref[...] = jnp.transpose(z)


OLB = 4


def _out_body(rows_ref, emb_ref):
    for p in range(OLB):
        x = rows_ref[1024 * p : 1024 * (p + 1), :]
        for q in range(4):
            xt = jnp.transpose(x[256 * q : 256 * (q + 1), :])
            for u in range(4):
                emb_ref[
                    p, :, 1024 * q + 256 * u : 1024 * q + 256 * (u + 1)
                ] = xt[32 * u : 32 * (u + 1), :]


def _make_gather(n_total: int, v_rows: int):
    assert n_total % W == 0
    mesh = plsc.VectorSubcoreMesh(core_axis_name="c", subcore_axis_name="s")

    @functools.partial(
        pl.kernel,
        mesh=mesh,
        out_type=jax.ShapeDtypeStruct((n_total, HID_C), jnp.float32),
        compiler_params=pltpu.CompilerParams(
            needs_layout_passes=False, use_tc_tiling_on_sc=False
        ),
        scratch_types=[
            pltpu.VMEM((G, GW), jnp.int32),
            pltpu.SemaphoreType.DMA,
        ],
    )
    def gather_kernel(idx_hbm, table_hbm, out_hbm, idx2, sem):
        def body(idx_vmem, out_vmem):
            for w in range(G):
                for k in range(GW // 16):
                    c = lax.iota(jnp.int32, 16) + 16 * k
                    rowv = 2 * (c % 4) + (w // 4)
                    colv = 32 * (w % 4) + c // 4
                    idx2[w, pl.ds(16 * k, 16)] = plsc.load_gather(
                        idx_vmem, [rowv, colv]
                    )
            copies = [
                pltpu.async_copy(
                    table_hbm.at[idx2.at[g]],
                    out_vmem.at[pl.ds(g * GW, GW)],
                    sem,
                )
                for g in range(G)
            ]
            for cp in copies:
                cp.wait()

        pltpu.emit_pipeline(
            body,
            grid=(n_total // W,),
            in_specs=[pl.BlockSpec((G, GW), index_map=lambda i: (i, 0))],
            out_specs=[pl.BlockSpec((W, HID_C), index_map=lambda i: (i, 0))],
            core_axis_name=("c", "s"),
            dimension_semantics=(pltpu.PARALLEL,),
        )(idx_hbm, out_hbm)

    return gather_kernel


@jax.jit
def kernel(source, target, table):
    b, seq = source.shape
    n_total = b * seq
    v_dim = table.shape[0]
    nvb = (v_dim + VC - 1) // VC

    src_t = source.T
    tgt_t = target.T

    idx_flat, t_t = pl.pallas_call(
        _prep_body,
        grid=(seq // LB,),
        in_specs=[
            pl.BlockSpec((LB, b), lambda i: (i, 0)),
            pl.BlockSpec((LB, b), lambda i: (i, 0)),
        ],
        out_specs=[
            pl.BlockSpec((LB * b // 128, 128), lambda i: (i, 0)),
            pl.BlockSpec((LB, b), lambda i: (i, 0)),
        ],
        out_shape=[
            jax.ShapeDtypeStruct((n_total // 128, 128), jnp.int32),
            jax.ShapeDtypeStruct((seq, b), target.dtype),
        ],
        compiler_params=pltpu.CompilerParams(
            dimension_semantics=("parallel",)
        ),
    )(src_t, tgt_t)

    w_tab = pl.pallas_call(
        _tab_body,
        grid=(nvb,),
        in_specs=[pl.BlockSpec((HID_C, VC), lambda i: (0, i))],
        out_specs=pl.BlockSpec((VC // 4, 128), lambda i: (i, 0)),
        out_shape=jax.ShapeDtypeStruct((nvb * VC // 4, 128), jnp.float32),
        compiler_params=pltpu.CompilerParams(
            dimension_semantics=("parallel",)
        ),
    )(table.T)

    rows = _make_gather(n_total, nvb * VC)(
        idx_flat, w_tab.reshape(nvb * VC, HID_C)
    )

    emb_t = pl.pallas_call(
        _out_body,
        grid=(seq // OLB,),
        in_specs=[pl.BlockSpec((OLB * b * HID_C // 128, 128), lambda i: (i, 0))],
        out_specs=pl.BlockSpec((OLB, HID_C, b), lambda i: (i, 0, 0)),
        out_shape=jax.ShapeDtypeStruct((seq, HID_C, b), jnp.float32),
        compiler_params=pltpu.CompilerParams(
            dimension_semantics=("parallel",)
        ),
    )(rows.reshape(n_total * HID_C // 128, 128))

    return (jnp.transpose(emb_t, (2, 0, 1)), t_t.T)


# speedup vs baseline: 4.8180x; 1.1676x over previous
import functools

import jax
import jax.numpy as jnp
from jax import lax
from jax.experimental import pallas as pl
from jax.experimental.pallas import tpu as pltpu
from jax.experimental.pallas import tpu_sc as plsc

IN_DIM = 1000000
HID_C = 32

GW = 128
G = 8
W = G * GW

LB = 8
VC = 16384


def _prep_body(src_ref, tgt_ref, idx_ref, t_ref):
    v = src_ref[...]
    s = v // 20 - 1
    s = jnp.where(s < 0, s + IN_DIM, s)
    rem = s % VC
    s = (s - rem) + 4 * (rem % (VC // 4)) + rem // (VC // 4)
    for l in range(LB):
        for q in range(32):
            idx_ref[32 * l + q : 32 * l + q + 1, :] = (
                s[l : l + 1, 128 * q : 128 * (q + 1)]
            )
    t_ref[...] = tgt_ref[...] // 20 - 1


def _tab_body(tab_ref, w_ref):
    x = tab_ref[...]
    z = jnp.concatenate(
        [x[:, (VC // 4) * u : (VC // 4) * (u + 1)] for u in range(4)], axis=0
    )
    w_ref[...] = jnp.transpose(z)


OLB = 8


def _out_body(rows_ref, emb_ref):
    for p in range(OLB):
        x = rows_ref[1024 * p : 1024 * (p + 1), :]
        for q in range(4):
            xt = jnp.transpose(x[256 * q : 256 * (q + 1), :])
            for u in range(4):
                emb_ref[
                    p, :, 1024 * q + 256 * u : 1024 * q + 256 * (u + 1)
                ] = xt[32 * u : 32 * (u + 1), :]


def _make_gather(n_total: int, v_rows: int):
    assert n_total % W == 0
    mesh = plsc.VectorSubcoreMesh(core_axis_name="c", subcore_axis_name="s")

    @functools.partial(
        pl.kernel,
        mesh=mesh,
        out_type=jax.ShapeDtypeStruct((n_total, HID_C), jnp.float32),
        compiler_params=pltpu.CompilerParams(
            needs_layout_passes=False, use_tc_tiling_on_sc=False
        ),
        scratch_types=[
            pltpu.VMEM((G, GW), jnp.int32),
            pltpu.SemaphoreType.DMA,
        ],
    )
    def gather_kernel(idx_hbm, table_hbm, out_hbm, idx2, sem):
        def body(idx_vmem, out_vmem):
            for w in range(G):
                for k in range(GW // 16):
                    c = lax.iota(jnp.int32, 16) + 16 * k
                    rowv = 2 * (c % 4) + (w // 4)
                    colv = 32 * (w % 4) + c // 4
                    idx2[w, pl.ds(16 * k, 16)] = plsc.load_gather(
                        idx_vmem, [rowv, colv]
                    )
            copies = [
                pltpu.async_copy(
                    table_hbm.at[idx2.at[g]],
                    out_vmem.at[pl.ds(g * GW, GW)],
                    sem,
                )
                for g in range(G)
            ]
            for cp in copies:
                cp.wait()

        pltpu.emit_pipeline(
            body,
            grid=(n_total // W,),
            in_specs=[pl.BlockSpec((G, GW), index_map=lambda i: (i, 0))],
            out_specs=[pl.BlockSpec((W, HID_C), index_map=lambda i: (i, 0))],
            core_axis_name=("c", "s"),
            dimension_semantics=(pltpu.PARALLEL,),
        )(idx_hbm, out_hbm)

    return gather_kernel


@jax.jit
def kernel(source, target, table):
    b, seq = source.shape
    n_total = b * seq
    v_dim = table.shape[0]
    nvb = (v_dim + VC - 1) // VC

    src_t = source.T
    tgt_t = target.T

    idx_flat, t_t = pl.pallas_call(
        _prep_body,
        grid=(seq // LB,),
        in_specs=[
            pl.BlockSpec((LB, b), lambda i: (i, 0)),
            pl.BlockSpec((LB, b), lambda i: (i, 0)),
        ],
        out_specs=[
            pl.BlockSpec((LB * b // 128, 128), lambda i: (i, 0)),
            pl.BlockSpec((LB, b), lambda i: (i, 0)),
        ],
        out_shape=[
            jax.ShapeDtypeStruct((n_total // 128, 128), jnp.int32),
            jax.ShapeDtypeStruct((seq, b), target.dtype),
        ],
        compiler_params=pltpu.CompilerParams(
            dimension_semantics=("parallel",)
        ),
    )(src_t, tgt_t)

    w_tab = pl.pallas_call(
        _tab_body,
        grid=(nvb,),
        in_specs=[pl.BlockSpec((HID_C, VC), lambda i: (0, i))],
        out_specs=pl.BlockSpec((VC // 4, 128), lambda i: (i, 0)),
        out_shape=jax.ShapeDtypeStruct((nvb * VC // 4, 128), jnp.float32),
        compiler_params=pltpu.CompilerParams(
            dimension_semantics=("parallel",)
        ),
    )(table.T)

    rows = _make_gather(n_total, nvb * VC)(
        idx_flat, w_tab.reshape(nvb * VC, HID_C)
    )

    emb_t = pl.pallas_call(
        _out_body,
        grid=(seq // OLB,),
        in_specs=[pl.BlockSpec((OLB * b * HID_C // 128, 128), lambda i: (i, 0))],
        out_specs=pl.BlockSpec((OLB, HID_C, b), lambda i: (i, 0, 0)),
        out_shape=jax.ShapeDtypeStruct((seq, HID_C, b), jnp.float32),
        compiler_params=pltpu.CompilerParams(
            dimension_semantics=("parallel",)
        ),
    )(rows.reshape(n_total * HID_C // 128, 128))

    return (jnp.transpose(emb_t, (2, 0, 1)), t_t.T)


# speedup vs baseline: 5.1524x; 1.0694x over previous
import functools

import jax
import jax.numpy as jnp
from jax import lax
from jax.experimental import pallas as pl
from jax.experimental.pallas import tpu as pltpu
from jax.experimental.pallas import tpu_sc as plsc

IN_DIM = 1000000
HID_C = 32

GW = 128
G = 8
W = G * GW

LB = 8
VC = 32768


def _prep_body(src_ref, tgt_ref, idx_ref, t_ref):
    v = src_ref[...]
    s = v // 20 - 1
    s = jnp.where(s < 0, s + IN_DIM, s)
    rem = s % VC
    s = (s - rem) + 4 * (rem % (VC // 4)) + rem // (VC // 4)
    for l in range(LB):
        for q in range(32):
            idx_ref[32 * l + q : 32 * l + q + 1, :] = (
                s[l : l + 1, 128 * q : 128 * (q + 1)]
            )
    t_ref[...] = tgt_ref[...] // 20 - 1


def _tab_body(tab_ref, w_ref):
    x = tab_ref[...]
    z = jnp.concatenate(
        [x[:, (VC // 4) * u : (VC // 4) * (u + 1)] for u in range(4)], axis=0
    )
    w_ref[...] = jnp.transpose(z)


OLB = 16


def _out_body(rows_ref, emb_ref):
    for p in range(OLB):
        x = rows_ref[1024 * p : 1024 * (p + 1), :]
        for q in range(4):
            xt = jnp.transpose(x[256 * q : 256 * (q + 1), :])
            for u in range(4):
                emb_ref[
                    p, :, 1024 * q + 256 * u : 1024 * q + 256 * (u + 1)
                ] = xt[32 * u : 32 * (u + 1), :]


def _make_gather(n_total: int, v_rows: int):
    assert n_total % W == 0
    mesh = plsc.VectorSubcoreMesh(core_axis_name="c", subcore_axis_name="s")

    @functools.partial(
        pl.kernel,
        mesh=mesh,
        out_type=jax.ShapeDtypeStruct((n_total, HID_C), jnp.float32),
        compiler_params=pltpu.CompilerParams(
            needs_layout_passes=False, use_tc_tiling_on_sc=False
        ),
        scratch_types=[
            pltpu.VMEM((G, GW), jnp.int32),
            pltpu.SemaphoreType.DMA,
        ],
    )
    def gather_kernel(idx_hbm, table_hbm, out_hbm, idx2, sem):
        def body(idx_vmem, out_vmem):
            for w in range(G):
                for k in range(GW // 16):
                    c = lax.iota(jnp.int32, 16) + 16 * k
                    rowv = 2 * (c % 4) + (w // 4)
                    colv = 32 * (w % 4) + c // 4
                    idx2[w, pl.ds(16 * k, 16)] = plsc.load_gather(
                        idx_vmem, [rowv, colv]
                    )
            copies = [
                pltpu.async_copy(
                    table_hbm.at[idx2.at[g]],
                    out_vmem.at[pl.ds(g * GW, GW)],
                    sem,
                )
                for g in range(G)
            ]
            for cp in copies:
                cp.wait()

        pltpu.emit_pipeline(
            body,
            grid=(n_total // W,),
            in_specs=[pl.BlockSpec((G, GW), index_map=lambda i: (i, 0))],
            out_specs=[pl.BlockSpec((W, HID_C), index_map=lambda i: (i, 0))],
            core_axis_name=("c", "s"),
            dimension_semantics=(pltpu.PARALLEL,),
        )(idx_hbm, out_hbm)

    return gather_kernel


@jax.jit
def kernel(source, target, table):
    b, seq = source.shape
    n_total = b * seq
    v_dim = table.shape[0]
    nvb = (v_dim + VC - 1) // VC

    src_t = source.T
    tgt_t = target.T

    idx_flat, t_t = pl.pallas_call(
        _prep_body,
        grid=(seq // LB,),
        in_specs=[
            pl.BlockSpec((LB, b), lambda i: (i, 0)),
            pl.BlockSpec((LB, b), lambda i: (i, 0)),
        ],
        out_specs=[
            pl.BlockSpec((LB * b // 128, 128), lambda i: (i, 0)),
            pl.BlockSpec((LB, b), lambda i: (i, 0)),
        ],
        out_shape=[
            jax.ShapeDtypeStruct((n_total // 128, 128), jnp.int32),
            jax.ShapeDtypeStruct((seq, b), target.dtype),
        ],
        compiler_params=pltpu.CompilerParams(
            dimension_semantics=("parallel",)
        ),
    )(src_t, tgt_t)

    w_tab = pl.pallas_call(
        _tab_body,
        grid=(nvb,),
        in_specs=[pl.BlockSpec((HID_C, VC), lambda i: (0, i))],
        out_specs=pl.BlockSpec((VC // 4, 128), lambda i: (i, 0)),
        out_shape=jax.ShapeDtypeStruct((nvb * VC // 4, 128), jnp.float32),
        compiler_params=pltpu.CompilerParams(
            dimension_semantics=("parallel",)
        ),
    )(table.T)

    rows = _make_gather(n_total, nvb * VC)(
        idx_flat, w_tab.reshape(nvb * VC, HID_C)
    )

    emb_t = pl.pallas_call(
        _out_body,
        grid=(seq // OLB,),
        in_specs=[pl.BlockSpec((OLB * b * HID_C // 128, 128), lambda i: (i, 0))],
        out_specs=pl.BlockSpec((OLB, HID_C, b), lambda i: (i, 0, 0)),
        out_shape=jax.ShapeDtypeStruct((seq, HID_C, b), jnp.float32),
        compiler_params=pltpu.CompilerParams(
            dimension_semantics=("parallel",)
        ),
    )(rows.reshape(n_total * HID_C // 128, 128))

    return (jnp.transpose(emb_t, (2, 0, 1)), t_t.T)
